# baseline (speedup 1.0000x reference)
import functools

import jax
import jax.numpy as jnp
from jax import lax
from jax.experimental import pallas as pl
from jax.experimental.pallas import tpu as pltpu
from jax.experimental.pallas import tpu_sc as plsc

EMB = 128
E = 320000
ROWS = E // EMB
NLUT = 128

NC = 2
NS = 16
NW = NC * NS
E_TC = 102400
E_SC = E - E_TC
TBLK = 512
EPW = E_SC // NW
CHUNK = 400
NCH = EPW // CHUNK


def _prep_body(w0_ref, w1_ref, w2_ref, w3_ref, lut_ref):
    c = lax.broadcasted_iota(jnp.int32, (NLUT, 1), 0)
    i0 = c // 24
    i1 = (c // 4) % 6
    i2 = (c // 2) % 2
    i3 = c % 2
    lut = jnp.zeros((NLUT, EMB), jnp.float32)
    for j in range(5):
        lut = lut + jnp.where(i0 == j, 1.0, 0.0) * w0_ref[j, :][None, :]
    for j in range(6):
        lut = lut + jnp.where(i1 == j, 1.0, 0.0) * w1_ref[j, :][None, :]
    for j in range(2):
        lut = lut + jnp.where(i2 == j, 1.0, 0.0) * w2_ref[j, :][None, :]
        lut = lut + jnp.where(i3 == j, 1.0, 0.0) * w3_ref[j, :][None, :]
    lut_ref[...] = lut


_prep = pl.pallas_call(
    _prep_body,
    out_shape=jax.ShapeDtypeStruct((NLUT, EMB), jnp.float32),
)

@functools.cache
def _make_sc_gather():
    mesh = plsc.VectorSubcoreMesh(core_axis_name="c", subcore_axis_name="s")

    @functools.partial(
        pl.kernel,
        mesh=mesh,
        out_type=jax.ShapeDtypeStruct((E, EMB), jnp.float32),
        scratch_types=[
            pltpu.VMEM((NLUT * EMB,), jnp.float32),
            pltpu.VMEM((CHUNK,), jnp.int32),
            pltpu.VMEM((CHUNK,), jnp.int32),
            pltpu.VMEM((CHUNK,), jnp.int32),
            pltpu.VMEM((CHUNK,), jnp.int32),
            pltpu.VMEM((CHUNK,), jnp.int32),
            pltpu.VMEM((CHUNK,), jnp.int32),
            pltpu.VMEM((CHUNK,), jnp.int32),
            pltpu.VMEM((CHUNK,), jnp.int32),
            pltpu.VMEM((CHUNK, EMB), jnp.float32),
            pltpu.VMEM((CHUNK, EMB), jnp.float32),
            pltpu.SemaphoreType.DMA((2,)),
            pltpu.SemaphoreType.DMA((2,)),
        ],
    )
    def _sc_gather(
        lut_hbm, a0_hbm, a1_hbm, a2_hbm, a3_hbm, out_hbm,
        lut_v, b00, b01, b02, b03, b10, b11, b12, b13,
        rows0, rows1, isem, ssem,
    ):
        ea_bufs = ((b00, b01, b02, b03), (b10, b11, b12, b13))
        col_hbm = (a0_hbm, a1_hbm, a2_hbm, a3_hbm)
        wid = lax.axis_index("s") * NC + lax.axis_index("c")
        base0 = wid * EPW

        def fetches(j, b):
            return [
                pltpu.make_async_copy(
                    col_hbm[t].at[pl.ds(base0 + j * CHUNK, CHUNK)],
                    ea_bufs[b][t],
                    isem.at[b],
                )
                for t in range(4)
            ]

        def fetch_start(j, b):
            for c in fetches(j, b):
                c.start()

        def fetch_wait(j, b):
            for c in fetches(j, b):
                c.wait()

        def compute(j, b, rows_ref):
            a0, a1, a2, a3 = ea_bufs[b]

            @plsc.parallel_loop(0, CHUNK // 16)
            def body(q):
                s = pl.ds(q * 16, 16)
                cvec = (
                    a0[s] * (24 * EMB)
                    + a1[s] * (4 * EMB)
                    + a2[s] * (2 * EMB)
                    + a3[s] * EMB
                )
                for l in range(16):
                    cb = cvec[l]
                    for k in range(8):
                        rows_ref[q * 16 + l, pl.ds(k * 16, 16)] = lut_v[
                            pl.ds(cb + k * 16, 16)
                        ]

        def scat(j, rows_ref, b):
            return pltpu.make_async_copy(
                rows_ref, out_hbm.at[pl.ds(base0 + j * CHUNK, CHUNK)], ssem.at[b]
            )

        fetch_start(0, 0)
        pltpu.sync_copy(lut_hbm, lut_v)

        def step(j, b, rows_ref, last):
            fetch_wait(j, b)
            if not last:
                fetch_start(j + 1, 1 - b)

            @pl.when(j >= 2)
            def _():
                scat(j - 2, rows_ref, b).wait()

            compute(j, b, rows_ref)
            scat(j, rows_ref, b).start()

        def pair(p, carry):
            j0 = 2 * p
            step(j0, 0, rows0, False)
            step(j0 + 1, 1, rows1, False)
            return carry

        lax.fori_loop(0, NCH // 2, pair, 0)

        step(NCH - 1, 0, rows0, True)
        scat(NCH - 2, rows1, 1).wait()
        scat(NCH - 1, rows0, 0).wait()

    return _sc_gather


def _tc_tail_body(prev_ref, ea_ref, lut_ref, out_ref):
    ea = ea_ref[...]
    combo = ea[:, 0] * 24 + ea[:, 1] * 4 + ea[:, 2] * 2 + ea[:, 3]
    cols = lax.broadcasted_iota(jnp.int32, (TBLK, NLUT), 1)
    oh = (combo[:, None] == cols).astype(jnp.float32)
    out_ref[...] = jax.lax.dot(oh, lut_ref[...], preferred_element_type=jnp.float32)


_tc_tail = pl.pallas_call(
    _tc_tail_body,
    grid=(E_TC // TBLK,),
    in_specs=[
        pl.BlockSpec((8, EMB), lambda i: (0, 0)),
        pl.BlockSpec((TBLK, 4), lambda i: (E_SC // TBLK + i, 0)),
        pl.BlockSpec((NLUT, EMB), lambda i: (0, 0)),
    ],
    out_specs=pl.BlockSpec((TBLK, EMB), lambda i: (E_SC // TBLK + i, 0)),
    out_shape=jax.ShapeDtypeStruct((E, EMB), jnp.float32),
    input_output_aliases={0: 0},
)


def kernel(edge_attr, W0, W1, W2, W3):
    lut = _prep(W0, W1, W2, W3)
    ea = edge_attr.astype(jnp.int32)
    cols = [ea[:, t] for t in range(4)]
    sc_out = _make_sc_gather()(lut.reshape(NLUT * EMB), *cols)
    return _tc_tail(sc_out, ea, lut)


# speedup vs baseline: 2.3873x; 2.3873x over previous
import functools

import jax
import jax.numpy as jnp
from jax import lax
from jax.experimental import pallas as pl
from jax.experimental.pallas import tpu as pltpu
from jax.experimental.pallas import tpu_sc as plsc

EMB = 128
E = 320000
ROWS = E // EMB
NLUT = 128

NC = 2
NS = 16
NW = NC * NS
EPW = E // NW
CHUNK = 400
NCH = EPW // CHUNK


def _prep_body(w0_ref, w1_ref, w2_ref, w3_ref, lut_ref):
    c = lax.broadcasted_iota(jnp.int32, (NLUT, 1), 0)
    i0 = c // 24
    i1 = (c // 4) % 6
    i2 = (c // 2) % 2
    i3 = c % 2
    lut = jnp.zeros((NLUT, EMB), jnp.float32)
    for j in range(5):
        lut = lut + jnp.where(i0 == j, 1.0, 0.0) * w0_ref[j, :][None, :]
    for j in range(6):
        lut = lut + jnp.where(i1 == j, 1.0, 0.0) * w1_ref[j, :][None, :]
    for j in range(2):
        lut = lut + jnp.where(i2 == j, 1.0, 0.0) * w2_ref[j, :][None, :]
        lut = lut + jnp.where(i3 == j, 1.0, 0.0) * w3_ref[j, :][None, :]
    lut_ref[...] = lut


_prep = pl.pallas_call(
    _prep_body,
    out_shape=jax.ShapeDtypeStruct((NLUT, EMB), jnp.float32),
)

@functools.cache
def _make_sc_gather():
    mesh = plsc.VectorSubcoreMesh(core_axis_name="c", subcore_axis_name="s")

    @functools.partial(
        pl.kernel,
        mesh=mesh,
        out_type=jax.ShapeDtypeStruct((E, EMB), jnp.float32),
        scratch_types=[
            pltpu.VMEM((NLUT * EMB,), jnp.float32),
            pltpu.VMEM((CHUNK,), jnp.int32),
            pltpu.VMEM((CHUNK,), jnp.int32),
            pltpu.VMEM((CHUNK,), jnp.int32),
            pltpu.VMEM((CHUNK,), jnp.int32),
            pltpu.VMEM((CHUNK,), jnp.int32),
            pltpu.VMEM((CHUNK,), jnp.int32),
            pltpu.VMEM((CHUNK,), jnp.int32),
            pltpu.VMEM((CHUNK,), jnp.int32),
            pltpu.VMEM((CHUNK, EMB), jnp.float32),
            pltpu.VMEM((CHUNK, EMB), jnp.float32),
            pltpu.SemaphoreType.DMA((2,)),
            pltpu.SemaphoreType.DMA((2,)),
        ],
    )
    def _sc_gather(
        lut_hbm, a0_hbm, a1_hbm, a2_hbm, a3_hbm, out_hbm,
        lut_v, b00, b01, b02, b03, b10, b11, b12, b13,
        rows0, rows1, isem, ssem,
    ):
        ea_bufs = ((b00, b01, b02, b03), (b10, b11, b12, b13))
        col_hbm = (a0_hbm, a1_hbm, a2_hbm, a3_hbm)
        wid = lax.axis_index("s") * NC + lax.axis_index("c")
        base0 = wid * EPW

        def fetches(j, b):
            return [
                pltpu.make_async_copy(
                    col_hbm[t].at[pl.ds(base0 + j * CHUNK, CHUNK)],
                    ea_bufs[b][t],
                    isem.at[b],
                )
                for t in range(4)
            ]

        def fetch_start(j, b):
            for c in fetches(j, b):
                c.start()

        def fetch_wait(j, b):
            for c in fetches(j, b):
                c.wait()

        def compute(j, b, rows_ref):
            a0, a1, a2, a3 = ea_bufs[b]

            @plsc.parallel_loop(0, CHUNK // 16)
            def body(q):
                s = pl.ds(q * 16, 16)
                cvec = (
                    a0[s] * (24 * EMB)
                    + a1[s] * (4 * EMB)
                    + a2[s] * (2 * EMB)
                    + a3[s] * EMB
                )
                for l in range(16):
                    cb = cvec[l]
                    for k in range(8):
                        rows_ref[q * 16 + l, pl.ds(k * 16, 16)] = lut_v[
                            pl.ds(cb + k * 16, 16)
                        ]

        def scat(j, rows_ref, b):
            return pltpu.make_async_copy(
                rows_ref, out_hbm.at[pl.ds(base0 + j * CHUNK, CHUNK)], ssem.at[b]
            )

        fetch_start(0, 0)
        pltpu.sync_copy(lut_hbm, lut_v)

        def step(j, b, rows_ref, last):
            fetch_wait(j, b)
            if not last:
                fetch_start(j + 1, 1 - b)

            @pl.when(j >= 2)
            def _():
                scat(j - 2, rows_ref, b).wait()

            compute(j, b, rows_ref)
            scat(j, rows_ref, b).start()

        def pair(p, carry):
            j0 = 2 * p
            step(j0, 0, rows0, False)
            step(j0 + 1, 1, rows1, False)
            return carry

        lax.fori_loop(0, NCH // 2, pair, 0)

        step(NCH - 1, 0, rows0, True)
        scat(NCH - 2, rows1, 1).wait()
        scat(NCH - 1, rows0, 0).wait()

    return _sc_gather


def kernel(edge_attr, W0, W1, W2, W3):
    lut = _prep(W0, W1, W2, W3)
    ea = edge_attr.astype(jnp.int32)
    cols = [ea[:, t] for t in range(4)]
    return _make_sc_gather()(lut.reshape(NLUT * EMB), *cols)
